# baseline (speedup 1.0000x reference)
import functools

import jax
import jax.numpy as jnp
from jax import lax
from jax.experimental import pallas as pl
from jax.experimental.pallas import tpu as pltpu
from jax.experimental.pallas import tpu_sc as plsc

VOCAB = 100000
D = 64
B = 16384

NC = 2
NS = 16
L = 16
NW = NC * NS
BPW = B // NW
CHUNK = 128
NCHUNK = BPW // CHUNK
NBLK = BPW // L


def _rsqrt16(x):
    i = lax.bitcast_convert_type(x, jnp.int32)
    i = jnp.int32(0x5F3759DF) - (i >> 1)
    y = lax.bitcast_convert_type(i, jnp.float32)
    half_x = x * 0.5
    for _ in range(3):
        y = y * (1.5 - half_x * y * y)
    return y


def _body(center_hbm, context_hbm, ctab_hbm, xtab_hbm, out_hbm,
          cidx_v, xidx_v, crows_v, xrows_v, cout_v, sem):
    wid = lax.axis_index("s") * NC + lax.axis_index("c")
    base = wid * BPW

    for j in range(NCHUNK):
        pltpu.sync_copy(center_hbm.at[pl.ds(base + j * CHUNK, CHUNK)],
                        cidx_v.at[j])
        pltpu.sync_copy(context_hbm.at[pl.ds(base + j * CHUNK, CHUNK)],
                        xidx_v.at[j])

    copies = []
    for j in range(NCHUNK):
        copies.append(pltpu.async_copy(
            ctab_hbm.at[cidx_v.at[j]],
            crows_v.at[pl.ds(j * CHUNK, CHUNK)], sem))
        copies.append(pltpu.async_copy(
            xtab_hbm.at[xidx_v.at[j]],
            xrows_v.at[pl.ds(j * CHUNK, CHUNK)], sem))
    for cp in copies:
        cp.wait()

    lane = lax.iota(jnp.int32, L)

    def blk(b, _):
        rowv = lane + b * L
        dot = jnp.zeros((L,), jnp.float32)
        cc = jnp.zeros((L,), jnp.float32)
        xx = jnp.zeros((L,), jnp.float32)
        for d in range(D):
            colv = jnp.full((L,), d, jnp.int32)
            cv = plsc.load_gather(crows_v, [rowv, colv])
            xv = plsc.load_gather(xrows_v, [rowv, colv])
            dot = dot + cv * xv
            cc = cc + cv * cv
            xx = xx + xv * xv
        m = jnp.maximum(cc, 1e-12) * jnp.maximum(xx, 1e-12)
        prob = (1.0 + dot * _rsqrt16(m)) * 0.5
        cout_v[pl.ds(b * L, L)] = prob
        return 0

    lax.fori_loop(0, NBLK, blk, 0)

    pltpu.sync_copy(cout_v, out_hbm.at[pl.ds(base, BPW)])


_sc_call = functools.partial(
    pl.kernel,
    out_type=jax.ShapeDtypeStruct((B,), jnp.float32),
    mesh=plsc.VectorSubcoreMesh(core_axis_name="c", subcore_axis_name="s",
                                num_cores=NC, num_subcores=NS),
    compiler_params=pltpu.CompilerParams(needs_layout_passes=False,
                                         use_tc_tiling_on_sc=False),
    scratch_types=[
        pltpu.VMEM((NCHUNK, CHUNK), jnp.int32),
        pltpu.VMEM((NCHUNK, CHUNK), jnp.int32),
        pltpu.VMEM((BPW, D), jnp.float32),
        pltpu.VMEM((BPW, D), jnp.float32),
        pltpu.VMEM((BPW,), jnp.float32),
        pltpu.SemaphoreType.DMA,
    ],
)(_body)


@jax.jit
def kernel(center, context, center_table, context_table):
    out = _sc_call(center, context, center_table, context_table)
    return out.reshape(B, 1)
